# baseline (speedup 1.0000x reference)
import functools

import jax
import jax.numpy as jnp
from jax import lax
from jax.experimental import pallas as pl
from jax.experimental.pallas import tpu as pltpu
from jax.experimental.pallas import tpu_sc as plsc


_DIM = 2048
_H = 8
_M = 1024
_DH = _DIM // _H
_TB = 256


def _tc_body(z_ref, z2_ref, c2_ref, cb_ref, idx_ref, gidx_ref, md_ref,
             cbt_ref):
    @pl.when(pl.program_id(0) == 0)
    def _():
        for h in range(_H):
            cbt_ref[h] = jnp.transpose(cb_ref[h])

    iot = jax.lax.broadcasted_iota(jnp.int32, (_TB, _M), 1)
    for h in range(_H):
        zb = z_ref[:, h * _DH:(h + 1) * _DH]
        cross = jax.lax.dot_general(
            zb, cbt_ref[h],
            dimension_numbers=(((1,), (0,)), ((), ())),
            preferred_element_type=jnp.float32)
        z2 = z2_ref[:, h:h + 1]
        c2 = c2_ref[h:h + 1, :]
        d = (z2 + c2) - 2.0 * cross
        d = jnp.maximum(d, 0.0)
        m = jnp.min(d, axis=1, keepdims=True)
        idx = jnp.min(jnp.where(d == m, iot, _M), axis=1)
        idx_ref[:, h:h + 1] = idx[:, None]
        gidx_ref[h:h + 1, :] = (idx + h * _M)[None, :]
        md_ref[:, h:h + 1] = m


def _tc_call(zr, z2, c2, codebook):
    N = zr.shape[0]
    grid = (N // _TB,)
    return pl.pallas_call(
        _tc_body,
        grid=grid,
        in_specs=[
            pl.BlockSpec((_TB, _DIM), lambda i: (i, 0)),
            pl.BlockSpec((_TB, _H), lambda i: (i, 0)),
            pl.BlockSpec((_H, _M), lambda i: (0, 0)),
            pl.BlockSpec((_H, _M, _DH), lambda i: (0, 0, 0)),
        ],
        out_specs=[
            pl.BlockSpec((_TB, _H), lambda i: (i, 0)),
            pl.BlockSpec((_H, _TB), lambda i: (0, i)),
            pl.BlockSpec((_TB, _H), lambda i: (i, 0)),
        ],
        out_shape=[
            jax.ShapeDtypeStruct((N, _H), jnp.int32),
            jax.ShapeDtypeStruct((_H, N), jnp.int32),
            jax.ShapeDtypeStruct((N, _H), jnp.float32),
        ],
        scratch_shapes=[pltpu.VMEM((_H, _DH, _M), jnp.float32)],
    )(zr, z2, c2, codebook)


_SC_CHUNK = 128
_SC_SEGS = 4


def _sc_gather(table, gidx):
    N = gidx.shape[1]
    toks_per_w = N // _SC_SEGS
    n_chunks = toks_per_w // _SC_CHUNK
    mesh = plsc.VectorSubcoreMesh(core_axis_name="c", subcore_axis_name="s")

    @functools.partial(
        pl.kernel, mesh=mesh,
        out_type=jax.ShapeDtypeStruct((N, _DIM), jnp.float32),
        scratch_types=[
            pltpu.VMEM((_SC_CHUNK,), jnp.int32),
            pltpu.VMEM((_SC_CHUNK, _DH), jnp.float32),
            pltpu.SemaphoreType.DMA,
        ],
    )
    def k(table_hbm, gidx_hbm, out_hbm, idx_v, rows_v, sem):
        wid = lax.axis_index("s") * 2 + lax.axis_index("c")
        h = wid % _H
        tok0 = (wid // _H) * toks_per_w
        for c in range(n_chunks):
            off = tok0 + c * _SC_CHUNK
            pltpu.sync_copy(gidx_hbm.at[h, pl.ds(off, _SC_CHUNK)], idx_v)
            pltpu.async_copy(table_hbm.at[idx_v], rows_v, sem).wait()
            pltpu.sync_copy(
                rows_v,
                out_hbm.at[pl.ds(off, _SC_CHUNK), pl.ds(h * _DH, _DH)])

    return k(table, gidx)


def kernel(z, codebook):
    Bb, Ll, dim = z.shape
    N = Bb * Ll
    zr = z.reshape(N, dim)
    z2 = jnp.stack(
        [jnp.sum(zr[:, h * _DH:(h + 1) * _DH] ** 2, axis=-1)
         for h in range(_H)], axis=-1)
    c2 = jnp.sum(codebook ** 2, axis=-1)

    idx, gidx, md = _tc_call(zr, z2, c2, codebook)

    table = codebook.reshape(_H * _M, _DH)
    zq = _sc_gather(table, gidx)

    return (zq.reshape(Bb, Ll, dim),
            idx.reshape(Bb, Ll, _H),
            md.reshape(Bb, Ll, _H))


# speedup vs baseline: 1.1499x; 1.1499x over previous
import functools

import jax
import jax.numpy as jnp
from jax import lax
from jax.experimental import pallas as pl
from jax.experimental.pallas import tpu as pltpu
from jax.experimental.pallas import tpu_sc as plsc


_DIM = 2048
_H = 8
_M = 1024
_DH = _DIM // _H
_TB = 256


def _tc_body(z_ref, c2_ref, cb_ref, idx_ref, gidx_ref, md_ref):
    iot = jax.lax.broadcasted_iota(jnp.int32, (_TB, _M), 1)
    for h in range(_H):
        zb = z_ref[:, h * _DH:(h + 1) * _DH]
        cross = jax.lax.dot_general(
            zb, cb_ref[h],
            dimension_numbers=(((1,), (1,)), ((), ())),
            preferred_element_type=jnp.float32)
        z2 = jnp.sum(zb * zb, axis=1, keepdims=True)
        c2 = c2_ref[h:h + 1, :]
        d = (z2 + c2) - 2.0 * cross
        d = jnp.maximum(d, 0.0)
        m = jnp.min(d, axis=1, keepdims=True)
        idx = jnp.min(jnp.where(d == m, iot, _M), axis=1)
        idx_ref[:, h:h + 1] = idx[:, None]
        gidx_ref[h:h + 1, :] = (idx + h * _M)[None, :]
        md_ref[:, h:h + 1] = m


def _tc_call(zr, c2, codebook):
    N = zr.shape[0]
    grid = (N // _TB,)
    return pl.pallas_call(
        _tc_body,
        grid=grid,
        in_specs=[
            pl.BlockSpec((_TB, _DIM), lambda i: (i, 0)),
            pl.BlockSpec((_H, _M), lambda i: (0, 0)),
            pl.BlockSpec((_H, _M, _DH), lambda i: (0, 0, 0)),
        ],
        out_specs=[
            pl.BlockSpec((_TB, _H), lambda i: (i, 0)),
            pl.BlockSpec((_H, _TB), lambda i: (0, i)),
            pl.BlockSpec((_TB, _H), lambda i: (i, 0)),
        ],
        out_shape=[
            jax.ShapeDtypeStruct((N, _H), jnp.int32),
            jax.ShapeDtypeStruct((_H, N), jnp.int32),
            jax.ShapeDtypeStruct((N, _H), jnp.float32),
        ],
    )(zr, c2, codebook)


_SC_CHUNK = 128
_SC_SEGS = 4


def _sc_gather(table, gidx):
    N = gidx.shape[1]
    toks_per_w = N // _SC_SEGS
    n_chunks = toks_per_w // _SC_CHUNK
    mesh = plsc.VectorSubcoreMesh(core_axis_name="c", subcore_axis_name="s")

    @functools.partial(
        pl.kernel, mesh=mesh,
        out_type=jax.ShapeDtypeStruct((N, _DIM), jnp.float32),
        scratch_types=[
            pltpu.VMEM((_SC_CHUNK,), jnp.int32),
            pltpu.VMEM((_SC_CHUNK, _DH), jnp.float32),
            pltpu.SemaphoreType.DMA,
        ],
    )
    def k(table_hbm, gidx_hbm, out_hbm, idx_v, rows_v, sem):
        wid = lax.axis_index("s") * 2 + lax.axis_index("c")
        h = wid % _H
        tok0 = (wid // _H) * toks_per_w
        for c in range(n_chunks):
            off = tok0 + c * _SC_CHUNK
            pltpu.sync_copy(gidx_hbm.at[h, pl.ds(off, _SC_CHUNK)], idx_v)
            pltpu.async_copy(table_hbm.at[idx_v], rows_v, sem).wait()
            pltpu.sync_copy(
                rows_v,
                out_hbm.at[pl.ds(off, _SC_CHUNK), pl.ds(h * _DH, _DH)])

    return k(table, gidx)


def kernel(z, codebook):
    Bb, Ll, dim = z.shape
    N = Bb * Ll
    zr = z.reshape(N, dim)
    c2 = jnp.sum(codebook ** 2, axis=-1)

    idx, gidx, md = _tc_call(zr, c2, codebook)

    table = codebook.reshape(_H * _M, _DH)
    zq = _sc_gather(table, gidx)

    return (zq.reshape(Bb, Ll, dim),
            idx.reshape(Bb, Ll, _H),
            md.reshape(Bb, Ll, _H))


# speedup vs baseline: 1.1844x; 1.0300x over previous
import functools

import jax
import jax.numpy as jnp
from jax import lax
from jax.experimental import pallas as pl
from jax.experimental.pallas import tpu as pltpu
from jax.experimental.pallas import tpu_sc as plsc


_DIM = 2048
_H = 8
_M = 1024
_DH = _DIM // _H
_TB = 512


def _tc_body(z_ref, c2_ref, cb_ref, idx_ref, gidx_ref, md_ref):
    iot = jax.lax.broadcasted_iota(jnp.int32, (_TB, _M), 1)
    for h in range(_H):
        zb = z_ref[:, h * _DH:(h + 1) * _DH]
        cross = jax.lax.dot_general(
            zb, cb_ref[h],
            dimension_numbers=(((1,), (1,)), ((), ())),
            preferred_element_type=jnp.float32)
        z2 = jnp.sum(zb * zb, axis=1, keepdims=True)
        c2 = c2_ref[h:h + 1, :]
        d = (z2 + c2) - 2.0 * cross
        d = jnp.maximum(d, 0.0)
        m = jnp.min(d, axis=1, keepdims=True)
        idx = jnp.min(jnp.where(d == m, iot, _M), axis=1)
        idx_ref[:, h:h + 1] = idx[:, None]
        gidx_ref[h:h + 1, :] = (idx + h * _M)[None, :]
        md_ref[:, h:h + 1] = m


def _tc_call(zr, c2, codebook):
    N = zr.shape[0]
    grid = (N // _TB,)
    return pl.pallas_call(
        _tc_body,
        grid=grid,
        in_specs=[
            pl.BlockSpec((_TB, _DIM), lambda i: (i, 0)),
            pl.BlockSpec((_H, _M), lambda i: (0, 0)),
            pl.BlockSpec((_H, _M, _DH), lambda i: (0, 0, 0)),
        ],
        out_specs=[
            pl.BlockSpec((_TB, _H), lambda i: (i, 0)),
            pl.BlockSpec((_H, _TB), lambda i: (0, i)),
            pl.BlockSpec((_TB, _H), lambda i: (i, 0)),
        ],
        out_shape=[
            jax.ShapeDtypeStruct((N, _H), jnp.int32),
            jax.ShapeDtypeStruct((_H, N), jnp.int32),
            jax.ShapeDtypeStruct((N, _H), jnp.float32),
        ],
    )(zr, c2, codebook)


_SC_CHUNK = 128
_SC_SEGS = 4


def _sc_gather(table, gidx):
    N = gidx.shape[1]
    toks_per_w = N // _SC_SEGS
    n_chunks = toks_per_w // _SC_CHUNK
    mesh = plsc.VectorSubcoreMesh(core_axis_name="c", subcore_axis_name="s")

    @functools.partial(
        pl.kernel, mesh=mesh,
        out_type=jax.ShapeDtypeStruct((N, _DIM), jnp.float32),
        scratch_types=[
            pltpu.VMEM((_SC_CHUNK,), jnp.int32),
            pltpu.VMEM((_SC_CHUNK, _DH), jnp.float32),
            pltpu.SemaphoreType.DMA,
        ],
    )
    def k(table_hbm, gidx_hbm, out_hbm, idx_v, rows_v, sem):
        wid = lax.axis_index("s") * 2 + lax.axis_index("c")
        h = wid % _H
        tok0 = (wid // _H) * toks_per_w
        for c in range(n_chunks):
            off = tok0 + c * _SC_CHUNK
            pltpu.sync_copy(gidx_hbm.at[h, pl.ds(off, _SC_CHUNK)], idx_v)
            pltpu.async_copy(table_hbm.at[idx_v], rows_v, sem).wait()
            pltpu.sync_copy(
                rows_v,
                out_hbm.at[pl.ds(off, _SC_CHUNK), pl.ds(h * _DH, _DH)])

    return k(table, gidx)


def kernel(z, codebook):
    Bb, Ll, dim = z.shape
    N = Bb * Ll
    zr = z.reshape(N, dim)
    c2 = jnp.sum(codebook ** 2, axis=-1)

    idx, gidx, md = _tc_call(zr, c2, codebook)

    table = codebook.reshape(_H * _M, _DH)
    zq = _sc_gather(table, gidx)

    return (zq.reshape(Bb, Ll, dim),
            idx.reshape(Bb, Ll, _H),
            md.reshape(Bb, Ll, _H))


# speedup vs baseline: 1.2284x; 1.0371x over previous
import functools

import jax
import jax.numpy as jnp
from jax import lax
from jax.experimental import pallas as pl
from jax.experimental.pallas import tpu as pltpu
from jax.experimental.pallas import tpu_sc as plsc


_DIM = 2048
_H = 8
_M = 1024
_DH = _DIM // _H
_TB = 1024


def _tc_body(z_ref, c2_ref, cb_ref, idx_ref, gidx_ref, md_ref):
    iot = jax.lax.broadcasted_iota(jnp.int32, (_TB, _M), 1)
    for h in range(_H):
        zb = z_ref[:, h * _DH:(h + 1) * _DH]
        cross = jax.lax.dot_general(
            zb, cb_ref[h],
            dimension_numbers=(((1,), (1,)), ((), ())),
            preferred_element_type=jnp.float32)
        z2 = jnp.sum(zb * zb, axis=1, keepdims=True)
        c2 = c2_ref[h:h + 1, :]
        d = (z2 + c2) - 2.0 * cross
        d = jnp.maximum(d, 0.0)
        m = jnp.min(d, axis=1, keepdims=True)
        idx = jnp.min(jnp.where(d == m, iot, _M), axis=1)
        idx_ref[:, h:h + 1] = idx[:, None]
        gidx_ref[h:h + 1, :] = (idx + h * _M)[None, :]
        md_ref[:, h:h + 1] = m


def _tc_call(zr, c2, codebook):
    N = zr.shape[0]
    grid = (N // _TB,)
    return pl.pallas_call(
        _tc_body,
        grid=grid,
        in_specs=[
            pl.BlockSpec((_TB, _DIM), lambda i: (i, 0)),
            pl.BlockSpec((_H, _M), lambda i: (0, 0)),
            pl.BlockSpec((_H, _M, _DH), lambda i: (0, 0, 0)),
        ],
        out_specs=[
            pl.BlockSpec((_TB, _H), lambda i: (i, 0)),
            pl.BlockSpec((_H, _TB), lambda i: (0, i)),
            pl.BlockSpec((_TB, _H), lambda i: (i, 0)),
        ],
        out_shape=[
            jax.ShapeDtypeStruct((N, _H), jnp.int32),
            jax.ShapeDtypeStruct((_H, N), jnp.int32),
            jax.ShapeDtypeStruct((N, _H), jnp.float32),
        ],
    )(zr, c2, codebook)


_SC_CHUNK = 128
_SC_SEGS = 4


def _sc_gather(table, gidx):
    N = gidx.shape[1]
    toks_per_w = N // _SC_SEGS
    n_chunks = toks_per_w // _SC_CHUNK
    mesh = plsc.VectorSubcoreMesh(core_axis_name="c", subcore_axis_name="s")

    @functools.partial(
        pl.kernel, mesh=mesh,
        out_type=jax.ShapeDtypeStruct((N, _DIM), jnp.float32),
        scratch_types=[
            pltpu.VMEM((_SC_CHUNK,), jnp.int32),
            pltpu.VMEM((_SC_CHUNK, _DH), jnp.float32),
            pltpu.SemaphoreType.DMA,
        ],
    )
    def k(table_hbm, gidx_hbm, out_hbm, idx_v, rows_v, sem):
        wid = lax.axis_index("s") * 2 + lax.axis_index("c")
        h = wid % _H
        tok0 = (wid // _H) * toks_per_w
        for c in range(n_chunks):
            off = tok0 + c * _SC_CHUNK
            pltpu.sync_copy(gidx_hbm.at[h, pl.ds(off, _SC_CHUNK)], idx_v)
            pltpu.async_copy(table_hbm.at[idx_v], rows_v, sem).wait()
            pltpu.sync_copy(
                rows_v,
                out_hbm.at[pl.ds(off, _SC_CHUNK), pl.ds(h * _DH, _DH)])

    return k(table, gidx)


def kernel(z, codebook):
    Bb, Ll, dim = z.shape
    N = Bb * Ll
    zr = z.reshape(N, dim)
    c2 = jnp.sum(codebook ** 2, axis=-1)

    idx, gidx, md = _tc_call(zr, c2, codebook)

    table = codebook.reshape(_H * _M, _DH)
    zq = _sc_gather(table, gidx)

    return (zq.reshape(Bb, Ll, dim),
            idx.reshape(Bb, Ll, _H),
            md.reshape(Bb, Ll, _H))
